# baseline (speedup 1.0000x reference)
import functools

import jax
import jax.numpy as jnp
import numpy as np
from jax.experimental import pallas as pl
from jax.experimental.pallas import tpu as pltpu

_BT = 96
_VMEM_LIMIT = 64 * 1024 * 1024

_H1, _W1 = 14, 14
_H3, _W3 = 7, 7
_WP2 = 18
_WP3 = 16
_WP4 = 9


def _mm(a, b):
    return jnp.dot(a, b, preferred_element_type=jnp.float32)


def _conv_stack_kernel(xeo_ref, w1_ref, w2_ref, w3_ref, w3s_ref, w4_ref,
                       w4s_ref, fcw_ref, fcb_ref, o_ref,
                       x2_ref, x3_ref, x4_ref):
    cdt = w1_ref.dtype
    bt = xeo_ref.shape[1] // 32

    def relu(x):
        return jnp.maximum(x, 0.0).astype(cdt)

    w1 = w1_ref[...]
    w2 = w2_ref[...]
    w3 = w3_ref[...]
    w4 = w4_ref[...]

    x2_ref[...] = jnp.zeros(x2_ref.shape, cdt)
    x3_ref[...] = jnp.zeros(x3_ref.shape, cdt)
    x4_ref[...] = jnp.zeros(x4_ref.shape, cdt)

    lhs1 = jnp.concatenate(
        [xeo_ref[0, (kh % 2) * 16 * bt + (kh // 2) * bt:
                 (kh % 2) * 16 * bt + (kh // 2) * bt + _H1 * bt, :]
         for kh in range(5)], axis=1)
    y1 = relu(_mm(lhs1, w1))
    for i in range(_H1):
        for j in range(_W1):
            x2_ref[((i + 2) * _WP2 + j + 2) * bt:
                   ((i + 2) * _WP2 + j + 3) * bt, :] = \
                y1[i * bt:(i + 1) * bt, j * 128:(j + 1) * 128]

    for p in range(7):
        i = 2 * p
        lhs = jnp.concatenate(
            [x2_ref[((i + kh) * _WP2 + kw) * bt:
                    ((i + kh) * _WP2 + kw) * bt + _W1 * bt, :]
             for kh in range(6) for kw in range(5)], axis=1)
        y = relu(_mm(lhs, w2))
        x3_ref[((i + 1) * _WP3 + 1) * bt:((i + 1) * _WP3 + 1 + _W1) * bt,
               :] = y[:, :128]
        x3_ref[((i + 2) * _WP3 + 1) * bt:((i + 2) * _WP3 + 1 + _W1) * bt,
               :] = y[:, 128:]

    def gather_even(y_half, i7):
        for j7 in range(_W3):
            x4_ref[((i7 + 1) * _WP4 + j7 + 1) * bt:
                   ((i7 + 1) * _WP4 + j7 + 2) * bt, :] = \
                y_half[2 * j7 * bt:(2 * j7 + 1) * bt, :]

    for p in range(3):
        lhs = jnp.concatenate(
            [x3_ref[((4 * p + kh) * _WP3 + kw) * bt:
                    ((4 * p + kh) * _WP3 + kw) * bt + _W1 * bt, :]
             for kh in range(5) for kw in range(3)], axis=1)
        y = relu(_mm(lhs, w3))
        gather_even(y[:, :128], 2 * p)
        gather_even(y[:, 128:], 2 * p + 1)
    lhs = jnp.concatenate(
        [x3_ref[((12 + kh) * _WP3 + kw) * bt:
                ((12 + kh) * _WP3 + kw) * bt + _W1 * bt, :]
         for kh in range(3) for kw in range(3)], axis=1)
    gather_even(relu(_mm(lhs, w3s_ref[...])), 6)

    feats = []
    for p in range(3):
        i7 = 2 * p
        lhs = jnp.concatenate(
            [x4_ref[((i7 + kh) * _WP4 + kw) * bt:
                    ((i7 + kh) * _WP4 + kw) * bt + _W3 * bt, :]
             for kh in range(4) for kw in range(3)], axis=1)
        y = relu(_mm(lhs, w4))
        feats.append(y[:, :128])
        feats.append(y[:, 128:])
    lhs = jnp.concatenate(
        [x4_ref[((6 + kh) * _WP4 + kw) * bt:
                ((6 + kh) * _WP4 + kw) * bt + _W3 * bt, :]
         for kh in range(3) for kw in range(3)], axis=1)
    feats.append(relu(_mm(lhs, w4s_ref[...])))

    zwide = jnp.concatenate(
        [feats[i7][j7 * bt:(j7 + 1) * bt, :]
         for i7 in range(_H3) for j7 in range(_W3)], axis=1)
    o_ref[...] = (_mm(zwide, fcw_ref[...]) + fcb_ref[...])[None]


def _conv_stack(xeo_blocks, w1w, w2p, w3p, w3s, w4p, w4s, fc_w, fc_b):
    nblk, m1, k1 = xeo_blocks.shape
    bt = m1 // 32
    consts = (w1w, w2p, w3p, w3s, w4p, w4s, fc_w, fc_b)
    in_specs = [pl.BlockSpec((1, m1, k1), lambda i: (i, 0, 0))]
    in_specs += [pl.BlockSpec(c.shape, lambda i: (0, 0)) for c in consts]
    flops = int(2.2e8) * bt * nblk
    bytes_accessed = int(xeo_blocks.size * 2 + nblk * bt * 49 * 128 * 2
                         + sum(c.size for c in consts) * 2)
    return pl.pallas_call(
        _conv_stack_kernel,
        out_shape=jax.ShapeDtypeStruct((nblk, bt, 128), jnp.float32),
        grid=(nblk,),
        in_specs=in_specs,
        out_specs=pl.BlockSpec((1, bt, 128), lambda i: (i, 0, 0)),
        scratch_shapes=[
            pltpu.VMEM((_WP2 * _WP2 * bt, 128), w1w.dtype),
            pltpu.VMEM((_WP3 * _WP3 * bt, 128), w1w.dtype),
            pltpu.VMEM((_WP4 * _WP4 * bt, 128), w1w.dtype),
        ],
        compiler_params=pltpu.CompilerParams(
            dimension_semantics=("parallel",),
            vmem_limit_bytes=_VMEM_LIMIT),
        cost_estimate=pl.CostEstimate(flops=flops, transcendentals=0,
                                      bytes_accessed=bytes_accessed),
    )(xeo_blocks, *consts)


def _fc_kernel(z_ref, w_ref, b_ref, o_ref):
    o_ref[...] = _mm(z_ref[...], w_ref[...]) + b_ref[...]


def _fc(z, fc_w, fc_b):
    bp, kin = z.shape
    od = fc_w.shape[1]
    tm = min(512, bp)
    return pl.pallas_call(
        _fc_kernel,
        out_shape=jax.ShapeDtypeStruct((bp, od), jnp.float32),
        grid=(pl.cdiv(bp, tm),),
        in_specs=[pl.BlockSpec((tm, kin), lambda i: (i, 0)),
                  pl.BlockSpec((kin, od), lambda i: (0, 0)),
                  pl.BlockSpec((1, od), lambda i: (0, 0))],
        out_specs=pl.BlockSpec((tm, od), lambda i: (i, 0)),
        compiler_params=pltpu.CompilerParams(
            dimension_semantics=("parallel",),
            vmem_limit_bytes=_VMEM_LIMIT),
    )(z, fc_w, fc_b)


def _quant_kernel(e_ref, s_ref, o_ref):
    s = s_ref[...]
    q = jnp.clip(jnp.round(e_ref[...] * s), -128.0, 127.0)
    o_ref[...] = q / s


def _quant(emb, scale):
    b, d = emb.shape
    tm = min(1024, b)
    return pl.pallas_call(
        _quant_kernel,
        out_shape=jax.ShapeDtypeStruct((b, d), jnp.float32),
        grid=(pl.cdiv(b, tm),),
        in_specs=[pl.BlockSpec((tm, d), lambda i: (i, 0)),
                  pl.BlockSpec((1, 1), lambda i: (0, 0))],
        out_specs=pl.BlockSpec((tm, d), lambda i: (i, 0)),
        compiler_params=pltpu.CompilerParams(dimension_semantics=("parallel",)),
    )(emb, scale)


def _pixel_planes(x_nchw, bt, cdt):
    b = x_nchw.shape[0]
    xp = jnp.pad(x_nchw.reshape(b, 28, 28), ((0, 0), (2, 2), (2, 2)))
    bp = -(-b // bt) * bt
    xp = jnp.pad(xp, ((0, bp - b), (0, 0), (0, 0)))
    planes = jnp.stack([xp[:, 0::2, :], xp[:, 1::2, :]], axis=1)
    planes = planes.reshape(bp // bt, bt, 2, 16, 32).transpose(0, 2, 3, 1, 4)
    return planes.reshape(bp // bt, 2 * 16 * bt, 32).astype(cdt), bp


def _conv1_weight(w1r, cdt):
    w1f = w1r[:25, :].astype(jnp.float32).reshape(5, 5, 128)
    wz = jnp.zeros((5, 32, _W1, 128), jnp.float32)
    j = np.arange(_W1)
    for kw in range(5):
        val = jnp.broadcast_to(w1f[:, kw, None, :], (5, _W1, 128))
        wz = wz.at[:, 2 * j + kw, j, :].set(val)
    return wz.reshape(5 * 32, _W1 * 128).astype(cdt)


def _pair_weight(wr, shift):
    left = jnp.pad(wr, ((0, shift), (0, 0)))
    right = jnp.pad(wr, ((shift, 0), (0, 0)))
    return jnp.concatenate([left, right], axis=1)


def _tperm(bt, dtype):
    n = bt * _H3 * _W3
    m = np.zeros((n, n), np.float32)
    b_idx = np.arange(n) // (_H3 * _W3)
    p_idx = np.arange(n) % (_H3 * _W3)
    m[np.arange(n), p_idx * bt + b_idx] = 1.0
    return jnp.asarray(m, dtype=dtype)


@jax.jit
def _forward(x_nchw, w1r, w2r, w3r, w4r, s12, s23, s34, s4v, fc_w, fc_b):
    b = x_nchw.shape[0]
    cdt = w1r.dtype
    xeo, bp = _pixel_planes(x_nchw, _BT, cdt)
    w2p = _pair_weight(w2r, 5 * 128)
    w3p = _pair_weight(w3r, 2 * 3 * 128)
    w4p = _pair_weight(w4r, 3 * 128)
    out = _conv_stack(xeo, _conv1_weight(w1r, cdt), w2p, w3p, w3r,
                      w4p, w4r, fc_w, fc_b)
    emb = out.reshape(bp, 128)[:b]
    abs_max = jnp.max(jnp.abs(emb))
    scale = (127.0 / (abs_max + 1e-8)).reshape(1, 1).astype(jnp.float32)
    return _quant(emb, scale)


def kernel(x_nchw, w1r, w2r, w3r, w4r, s12, s23, s34, s4v, fc_w, fc_b):
    return _forward(x_nchw, w1r, w2r, w3r, w4r, s12, s23, s34, s4v,
                    fc_w, fc_b)


# speedup vs baseline: 1.0294x; 1.0294x over previous
import jax
import jax.numpy as jnp
import numpy as np
from jax.experimental import pallas as pl
from jax.experimental.pallas import tpu as pltpu

_BT = 64
_VMEM_LIMIT = 64 * 1024 * 1024

_H1, _W1 = 14, 14
_H3, _W3 = 7, 7
_WP2 = 18
_WP3 = 16
_WP4 = 9


def _mm(a, b):
    return jnp.dot(a, b, preferred_element_type=jnp.float32)


def _conv_stack_kernel(xeo_ref, w1_ref, w2_ref, w3_ref, w3s_ref, w4_ref,
                       w4s_ref, fcw_ref, fcb_ref, o_ref,
                       x2_ref, x3_ref, x4_ref):
    cdt = w1_ref.dtype
    bt = xeo_ref.shape[1] // 32

    def relu(x):
        return jnp.maximum(x, 0.0).astype(cdt)

    w1 = w1_ref[...]
    w2 = w2_ref[...]
    w3 = w3_ref[...]
    w4 = w4_ref[...]

    x2_ref[...] = jnp.zeros(x2_ref.shape, cdt)
    x3_ref[...] = jnp.zeros(x3_ref.shape, cdt)
    x4_ref[...] = jnp.zeros(x4_ref.shape, cdt)

    lhs1 = jnp.concatenate(
        [xeo_ref[0, (kh % 2) * 16 * bt + (kh // 2) * bt:
                 (kh % 2) * 16 * bt + (kh // 2) * bt + _H1 * bt, :]
         for kh in range(5)], axis=1)
    y1 = relu(_mm(lhs1, w1))
    for i in range(_H1):
        for j in range(_W1):
            x2_ref[((i + 2) * _WP2 + j + 2) * bt:
                   ((i + 2) * _WP2 + j + 3) * bt, :] = \
                y1[i * bt:(i + 1) * bt, j * 128:(j + 1) * 128]

    for p in range(7):
        i = 2 * p
        lhs = jnp.concatenate(
            [x2_ref[((i + kh) * _WP2 + kw) * bt:
                    ((i + kh) * _WP2 + kw) * bt + _W1 * bt, :]
             for kh in range(6) for kw in range(5)], axis=1)
        y = relu(_mm(lhs, w2))
        x3_ref[((i + 1) * _WP3 + 1) * bt:((i + 1) * _WP3 + 1 + _W1) * bt,
               :] = y[:, :128]
        x3_ref[((i + 2) * _WP3 + 1) * bt:((i + 2) * _WP3 + 1 + _W1) * bt,
               :] = y[:, 128:]

    def gather_even(y_half, i7):
        for j7 in range(_W3):
            x4_ref[((i7 + 1) * _WP4 + j7 + 1) * bt:
                   ((i7 + 1) * _WP4 + j7 + 2) * bt, :] = \
                y_half[2 * j7 * bt:(2 * j7 + 1) * bt, :]

    for p in range(3):
        lhs = jnp.concatenate(
            [x3_ref[((4 * p + kh) * _WP3 + kw) * bt:
                    ((4 * p + kh) * _WP3 + kw) * bt + _W1 * bt, :]
             for kh in range(5) for kw in range(3)], axis=1)
        y = relu(_mm(lhs, w3))
        gather_even(y[:, :128], 2 * p)
        gather_even(y[:, 128:], 2 * p + 1)
    lhs = jnp.concatenate(
        [x3_ref[((12 + kh) * _WP3 + kw) * bt:
                ((12 + kh) * _WP3 + kw) * bt + _W1 * bt, :]
         for kh in range(3) for kw in range(3)], axis=1)
    gather_even(relu(_mm(lhs, w3s_ref[...])), 6)

    feats = []
    for p in range(3):
        i7 = 2 * p
        lhs = jnp.concatenate(
            [x4_ref[((i7 + kh) * _WP4 + kw) * bt:
                    ((i7 + kh) * _WP4 + kw) * bt + _W3 * bt, :]
             for kh in range(4) for kw in range(3)], axis=1)
        y = relu(_mm(lhs, w4))
        feats.append(y[:, :128])
        feats.append(y[:, 128:])
    lhs = jnp.concatenate(
        [x4_ref[((6 + kh) * _WP4 + kw) * bt:
                ((6 + kh) * _WP4 + kw) * bt + _W3 * bt, :]
         for kh in range(3) for kw in range(3)], axis=1)
    feats.append(relu(_mm(lhs, w4s_ref[...])))

    zwide = jnp.concatenate(
        [feats[i7][j7 * bt:(j7 + 1) * bt, :]
         for i7 in range(_H3) for j7 in range(_W3)], axis=1)
    o_ref[...] = (_mm(zwide, fcw_ref[...]) + fcb_ref[...])[None]


def _conv_stack(xeo_blocks, w1w, w2p, w3p, w3s, w4p, w4s, fc_w, fc_b):
    nblk, m1, k1 = xeo_blocks.shape
    bt = m1 // 32
    consts = (w1w, w2p, w3p, w3s, w4p, w4s, fc_w, fc_b)
    in_specs = [pl.BlockSpec((1, m1, k1), lambda i: (i, 0, 0))]
    in_specs += [pl.BlockSpec(c.shape, lambda i: (0, 0)) for c in consts]
    flops = int(2.2e8) * bt * nblk
    bytes_accessed = int(xeo_blocks.size * 2 + nblk * bt * 49 * 128 * 2
                         + sum(c.size for c in consts) * 2)
    return pl.pallas_call(
        _conv_stack_kernel,
        out_shape=jax.ShapeDtypeStruct((nblk, bt, 128), jnp.float32),
        grid=(nblk,),
        in_specs=in_specs,
        out_specs=pl.BlockSpec((1, bt, 128), lambda i: (i, 0, 0)),
        scratch_shapes=[
            pltpu.VMEM((_WP2 * _WP2 * bt, 128), w1w.dtype),
            pltpu.VMEM((_WP3 * _WP3 * bt, 128), w1w.dtype),
            pltpu.VMEM((_WP4 * _WP4 * bt, 128), w1w.dtype),
        ],
        compiler_params=pltpu.CompilerParams(
            dimension_semantics=("parallel",),
            vmem_limit_bytes=_VMEM_LIMIT),
        cost_estimate=pl.CostEstimate(flops=flops, transcendentals=0,
                                      bytes_accessed=bytes_accessed),
    )(xeo_blocks, *consts)


def _quant_kernel(e_ref, s_ref, o_ref):
    s = s_ref[...]
    q = jnp.clip(jnp.round(e_ref[...] * s), -128.0, 127.0)
    o_ref[...] = q / s


def _quant(emb, scale):
    b, d = emb.shape
    tm = min(1024, b)
    return pl.pallas_call(
        _quant_kernel,
        out_shape=jax.ShapeDtypeStruct((b, d), jnp.float32),
        grid=(pl.cdiv(b, tm),),
        in_specs=[pl.BlockSpec((tm, d), lambda i: (i, 0)),
                  pl.BlockSpec((1, 1), lambda i: (0, 0))],
        out_specs=pl.BlockSpec((tm, d), lambda i: (i, 0)),
        compiler_params=pltpu.CompilerParams(dimension_semantics=("parallel",)),
    )(emb, scale)


def _pixel_planes(x_nchw, bt, cdt):
    b = x_nchw.shape[0]
    xp = jnp.pad(x_nchw.reshape(b, 28, 28), ((0, 0), (2, 2), (2, 2)))
    bp = -(-b // bt) * bt
    xp = jnp.pad(xp, ((0, bp - b), (0, 0), (0, 0)))
    planes = jnp.stack([xp[:, 0::2, :], xp[:, 1::2, :]], axis=1)
    planes = planes.reshape(bp // bt, bt, 2, 16, 32).transpose(0, 2, 3, 1, 4)
    return planes.reshape(bp // bt, 2 * 16 * bt, 32).astype(cdt), bp


def _conv1_weight(w1r, cdt):
    w1f = w1r[:25, :].astype(jnp.float32).reshape(5, 5, 128)
    wz = jnp.zeros((5, 32, _W1, 128), jnp.float32)
    j = np.arange(_W1)
    for kw in range(5):
        val = jnp.broadcast_to(w1f[:, kw, None, :], (5, _W1, 128))
        wz = wz.at[:, 2 * j + kw, j, :].set(val)
    return wz.reshape(5 * 32, _W1 * 128).astype(cdt)


def _pair_weight(wr, shift):
    left = jnp.pad(wr, ((0, shift), (0, 0)))
    right = jnp.pad(wr, ((shift, 0), (0, 0)))
    return jnp.concatenate([left, right], axis=1)


@jax.jit
def _forward(x_nchw, w1r, w2r, w3r, w4r, s12, s23, s34, s4v, fc_w, fc_b):
    b = x_nchw.shape[0]
    cdt = w1r.dtype
    xeo, bp = _pixel_planes(x_nchw, _BT, cdt)
    w2p = _pair_weight(w2r, 5 * 128)
    w3p = _pair_weight(w3r, 2 * 3 * 128)
    w4p = _pair_weight(w4r, 3 * 128)
    out = _conv_stack(xeo, _conv1_weight(w1r, cdt), w2p, w3p, w3r,
                      w4p, w4r, fc_w, fc_b)
    emb = out.reshape(bp, 128)[:b]
    abs_max = jnp.max(jnp.abs(emb))
    scale = (127.0 / (abs_max + 1e-8)).reshape(1, 1).astype(jnp.float32)
    return _quant(emb, scale)


def kernel(x_nchw, w1r, w2r, w3r, w4r, s12, s23, s34, s4v, fc_w, fc_b):
    return _forward(x_nchw, w1r, w2r, w3r, w4r, s12, s23, s34, s4v,
                    fc_w, fc_b)
